# baseline (speedup 1.0000x reference)
import functools

import jax
import jax.numpy as jnp
from jax import lax
from jax.experimental import pallas as pl
from jax.experimental.pallas import tpu as pltpu
from jax.experimental.pallas import tpu_sc as plsc

_MAX_DIST2 = 0.25

_VP = 7168
_VPT = 6912
_FP = 13824
_FT = _FP // 16
_ROWS, _COLS = 168, 128
_TQ = 1024


def _sc_normals_body(vx_h, vy_h, vz_h, f0_h, f1_h, f2_h, z_h, r_h, out_h,
                     vx_v, vy_v, vz_v, acc_v, f0_v, f1_v, f2_v, r_v,
                     shared_v):
    bs = vx_h.shape[0] // _VP
    arows = bs * _ROWS
    cid = lax.axis_index("c")
    sid = lax.axis_index("s")
    base = (cid * 8 + sid) * _FT

    pltpu.sync_copy(r_h, r_v)
    pltpu.sync_copy(z_h, acc_v)

    @pl.when(sid < 8)
    def _():
      for b in range(bs):
        pltpu.sync_copy(vx_h.at[pl.ds(b * _VP, _VP)], vx_v)
        pltpu.sync_copy(vy_h.at[pl.ds(b * _VP, _VP)], vy_v)
        pltpu.sync_copy(vz_h.at[pl.ds(b * _VP, _VP)], vz_v)
        pltpu.sync_copy(f0_h.at[pl.ds(b * _FP + base, _FT)], f0_v)
        pltpu.sync_copy(f1_h.at[pl.ds(b * _FP + base, _FT)], f1_v)
        pltpu.sync_copy(f2_h.at[pl.ds(b * _FP + base, _FT)], f2_v)

        def _faces(j, carry):
            i0 = f0_v[pl.ds(j * 16, 16)]
            i1 = f1_v[pl.ds(j * 16, 16)]
            i2 = f2_v[pl.ds(j * 16, 16)]
            v0x = plsc.load_gather(vx_v, [i0])
            v0y = plsc.load_gather(vy_v, [i0])
            v0z = plsc.load_gather(vz_v, [i0])
            v1x = plsc.load_gather(vx_v, [i1])
            v1y = plsc.load_gather(vy_v, [i1])
            v1z = plsc.load_gather(vz_v, [i1])
            v2x = plsc.load_gather(vx_v, [i2])
            v2y = plsc.load_gather(vy_v, [i2])
            v2z = plsc.load_gather(vz_v, [i2])
            e1x, e1y, e1z = v1x - v0x, v1y - v0y, v1z - v0z
            e2x, e2y, e2z = v2x - v0x, v2y - v0y, v2z - v0z
            fnx = e1y * e2z - e1z * e2y
            fny = e1z * e2x - e1x * e2z
            fnz = e1x * e2y - e1y * e2x
            for comp, fv in ((0, fnx), (1, fny), (2, fnz)):
                for idx in (i0, i1, i2):
                    flat = idx + (b * 3 + comp) * _VP
                    plsc.addupdate_scatter(
                        acc_v, [flat // _COLS, flat % _COLS], fv)
            return carry
        lax.fori_loop(0, _FT // 16, _faces, 0)

    @pl.when(sid == 0)
    def _():
        pltpu.sync_copy(acc_v, shared_v)

    plsc.subcore_barrier()

    @pl.when((sid != 0) & (sid < 8))
    def _():
        nch, csz = r_h.shape
        for c in range(nch):
            pltpu.sync_copy(acc_v.at[pl.ds(c * csz, csz)],
                            shared_v.at[r_v.at[c]], add=True)

    plsc.subcore_barrier()

    @pl.when(sid == 0)
    def _():
        for b in range(bs):
            pltpu.sync_copy(shared_v.at[pl.ds(b * _ROWS, _ROWS)],
                            out_h.at[b * 2 + cid])


def _sc_normals(vx, vy, vz, f0, f1, f2):
    bs = vx.shape[0] // _VP
    arows = bs * _ROWS
    zeros = jnp.zeros((arows, _COLS), jnp.float32)
    csz = max(c for c in range(8, 129, 8) if arows % c == 0)
    mesh = plsc.VectorSubcoreMesh(core_axis_name="c", subcore_axis_name="s",
                                  num_cores=2, num_subcores=16)
    fn = pl.kernel(
        _sc_normals_body,
        out_type=jax.ShapeDtypeStruct((bs * 2, _ROWS, _COLS), jnp.float32),
        mesh=mesh,
        compiler_params=pltpu.CompilerParams(needs_layout_passes=False),
        scratch_types=[
            pltpu.VMEM((_VP,), jnp.float32),
            pltpu.VMEM((_VP,), jnp.float32),
            pltpu.VMEM((_VP,), jnp.float32),
            pltpu.VMEM((arows, _COLS), jnp.float32),
            pltpu.VMEM((_FT,), jnp.int32),
            pltpu.VMEM((_FT,), jnp.int32),
            pltpu.VMEM((_FT,), jnp.int32),
            pltpu.VMEM((arows // csz, csz), jnp.int32),
            pltpu.VMEM_SHARED((arows, _COLS), jnp.float32),
        ],
    )
    rows = jnp.arange(arows, dtype=jnp.int32).reshape(arows // csz, csz)
    return fn(vx, vy, vz, f0, f1, f2, zeros, rows)


def _tc_body(q_ref, a_ref, vn_ref, out_ref, *, nt):
    i = pl.program_id(1)
    a3 = a_ref[0, 0:3]
    vnp = vn_ref[0]
    vn3 = vnp[0:3] + vnp[8:11]
    q = q_ref[0]
    dx = q[:, 0:1] - a3[0:1, :]
    dy = q[:, 1:2] - a3[1:2, :]
    dz = q[:, 2:3] - a3[2:3, :]
    d2 = dx * dx + dy * dy + dz * dz
    m = jnp.min(d2, axis=1, keepdims=True)
    sel = jnp.where(d2 == m, 1.0, 0.0)
    an_t = jnp.concatenate([a3, vn3], axis=0)
    ansel = lax.dot_general(sel, an_t, (((1,), (1,)), ((), ())),
                            preferred_element_type=jnp.float32)
    diff = q[:, 0:3] - ansel[:, 0:3]
    dotv = jnp.sum(diff * ansel[:, 3:6], axis=1, keepdims=True)
    l2 = jnp.sqrt(m + 1e-20)
    coll = jnp.where((l2 <= 0.5) & (dotv < 0.0), 1.0, 0.0)
    cnt = jnp.sum(coll)

    @pl.when(i == 0)
    def _():
        out_ref[0, 0, 0] = 0.0
        out_ref[0, 1, 0] = 0.0

    @pl.when(i < nt // 2)
    def _():
        out_ref[0, 0, 0] += cnt

    @pl.when(i >= nt // 2)
    def _():
        out_ref[0, 1, 0] += cnt


def _tc_collisions(q, a_t, vnp):
    bs, ng = q.shape[0], q.shape[1]
    nt = ng // _TQ
    return pl.pallas_call(
        functools.partial(_tc_body, nt=nt),
        grid=(bs, nt),
        in_specs=[
            pl.BlockSpec((1, _TQ, 8), lambda b, i: (b, i, 0)),
            pl.BlockSpec((1, 8, _VPT), lambda b, i: (b, 0, 0)),
            pl.BlockSpec((1, 16, _VPT), lambda b, i: (b, 0, 0)),
        ],
        out_specs=pl.BlockSpec((1, 2, 1), lambda b, i: (b, 0, 0),
                               memory_space=pltpu.SMEM),
        out_shape=jax.ShapeDtypeStruct((bs, 2, 1), jnp.float32),
    )(q, a_t, vnp)[:, :, 0]


def kernel(pred, target, indices, indices_type, faces, h_state, h_faces):
    bs, ng = pred.shape[0], pred.shape[1]
    nh = h_state.shape[1]

    q = jnp.concatenate(
        [pred.astype(jnp.float32), jnp.zeros((bs, ng, 5), jnp.float32)],
        axis=-1)
    a = jnp.transpose(h_state[:, :, :3].astype(jnp.float32), (0, 2, 1))
    a_pad = jnp.pad(a, ((0, 0), (0, 0), (0, _VPT - nh)),
                    constant_values=1e4)
    a_t = jnp.pad(a_pad, ((0, 0), (0, 5), (0, 0)))
    v_pad = jnp.pad(a, ((0, 0), (0, 0), (0, _VP - nh)))
    vx = v_pad[:, 0].reshape(-1)
    vy = v_pad[:, 1].reshape(-1)
    vz = v_pad[:, 2].reshape(-1)
    f_t = jnp.transpose(h_faces, (0, 2, 1)).astype(jnp.int32)
    f_t = jnp.pad(f_t, ((0, 0), (0, 0), (0, _FP - f_t.shape[2])),
                  constant_values=nh)
    f0 = f_t[:, 0].reshape(-1)
    f1 = f_t[:, 1].reshape(-1)
    f2 = f_t[:, 2].reshape(-1)

    vparts = _sc_normals(vx, vy, vz, f0, f1, f2)
    vparts = vparts.reshape(bs, 2, 3, _VP)[:, :, :, :_VPT]
    vparts = jnp.pad(vparts, ((0, 0), (0, 0), (0, 5), (0, 0)))
    vnp = vparts.reshape(bs, 16, _VPT)

    counts = _tc_collisions(q, a_t, vnp)

    total = indices[:, -1].astype(jnp.float32)
    first, second = counts[:, 0], counts[:, 1]
    g2h = jnp.mean((first + second) / total)
    top = jnp.mean(first * indices_type[:, 0, 0] / total)
    bottom = jnp.mean(second * indices_type[:, 1, 1] / total)
    return jnp.stack([bottom, g2h, top])


# speedup vs baseline: 1.0041x; 1.0041x over previous
import functools

import jax
import jax.numpy as jnp
from jax import lax
from jax.experimental import pallas as pl
from jax.experimental.pallas import tpu as pltpu
from jax.experimental.pallas import tpu_sc as plsc

_MAX_DIST2 = 0.25

_VP = 7168
_VPT = 6912
_FP = 13824
_FT = _FP // 32
_ROWS, _COLS = 168, 128
_TQ = 1024


def _sc_normals_body(vx_h, vy_h, vz_h, f0_h, f1_h, f2_h, z_h, r_h, out_h,
                     vx_v, vy_v, vz_v, acc_v, f0_v, f1_v, f2_v, r_v,
                     shared_v):
    bs = vx_h.shape[0] // _VP
    arows = bs * _ROWS
    cid = lax.axis_index("c")
    sid = lax.axis_index("s")
    base = (cid * 16 + sid) * _FT

    pltpu.sync_copy(r_h, r_v)
    pltpu.sync_copy(z_h, acc_v)

    for b in range(bs):
        pltpu.sync_copy(vx_h.at[pl.ds(b * _VP, _VP)], vx_v)
        pltpu.sync_copy(vy_h.at[pl.ds(b * _VP, _VP)], vy_v)
        pltpu.sync_copy(vz_h.at[pl.ds(b * _VP, _VP)], vz_v)
        pltpu.sync_copy(f0_h.at[pl.ds(b * _FP + base, _FT)], f0_v)
        pltpu.sync_copy(f1_h.at[pl.ds(b * _FP + base, _FT)], f1_v)
        pltpu.sync_copy(f2_h.at[pl.ds(b * _FP + base, _FT)], f2_v)

        def _faces(j, carry):
            i0 = f0_v[pl.ds(j * 16, 16)]
            i1 = f1_v[pl.ds(j * 16, 16)]
            i2 = f2_v[pl.ds(j * 16, 16)]
            v0x = plsc.load_gather(vx_v, [i0])
            v0y = plsc.load_gather(vy_v, [i0])
            v0z = plsc.load_gather(vz_v, [i0])
            v1x = plsc.load_gather(vx_v, [i1])
            v1y = plsc.load_gather(vy_v, [i1])
            v1z = plsc.load_gather(vz_v, [i1])
            v2x = plsc.load_gather(vx_v, [i2])
            v2y = plsc.load_gather(vy_v, [i2])
            v2z = plsc.load_gather(vz_v, [i2])
            e1x, e1y, e1z = v1x - v0x, v1y - v0y, v1z - v0z
            e2x, e2y, e2z = v2x - v0x, v2y - v0y, v2z - v0z
            fnx = e1y * e2z - e1z * e2y
            fny = e1z * e2x - e1x * e2z
            fnz = e1x * e2y - e1y * e2x
            for comp, fv in ((0, fnx), (1, fny), (2, fnz)):
                for idx in (i0, i1, i2):
                    flat = idx + (b * 3 + comp) * _VP
                    plsc.addupdate_scatter(
                        acc_v, [flat // _COLS, flat % _COLS], fv)
            return carry
        lax.fori_loop(0, _FT // 16, _faces, 0)

    @pl.when(sid == 0)
    def _():
        pltpu.sync_copy(acc_v, shared_v)

    plsc.subcore_barrier()

    @pl.when(sid != 0)
    def _():
        nch, csz = r_h.shape
        for c in range(nch):
            pltpu.sync_copy(acc_v.at[pl.ds(c * csz, csz)],
                            shared_v.at[r_v.at[c]], add=True)

    plsc.subcore_barrier()

    @pl.when(sid == 0)
    def _():
        for b in range(bs):
            pltpu.sync_copy(shared_v.at[pl.ds(b * _ROWS, _ROWS)],
                            out_h.at[b * 2 + cid])


def _sc_normals(vx, vy, vz, f0, f1, f2):
    bs = vx.shape[0] // _VP
    arows = bs * _ROWS
    zeros = jnp.zeros((arows, _COLS), jnp.float32)
    csz = max(c for c in range(8, 129, 8) if arows % c == 0)
    mesh = plsc.VectorSubcoreMesh(core_axis_name="c", subcore_axis_name="s",
                                  num_cores=2, num_subcores=16)
    fn = pl.kernel(
        _sc_normals_body,
        out_type=jax.ShapeDtypeStruct((bs * 2, _ROWS, _COLS), jnp.float32),
        mesh=mesh,
        compiler_params=pltpu.CompilerParams(needs_layout_passes=False),
        scratch_types=[
            pltpu.VMEM((_VP,), jnp.float32),
            pltpu.VMEM((_VP,), jnp.float32),
            pltpu.VMEM((_VP,), jnp.float32),
            pltpu.VMEM((arows, _COLS), jnp.float32),
            pltpu.VMEM((_FT,), jnp.int32),
            pltpu.VMEM((_FT,), jnp.int32),
            pltpu.VMEM((_FT,), jnp.int32),
            pltpu.VMEM((arows // csz, csz), jnp.int32),
            pltpu.VMEM_SHARED((arows, _COLS), jnp.float32),
        ],
    )
    rows = jnp.arange(arows, dtype=jnp.int32).reshape(arows // csz, csz)
    return fn(vx, vy, vz, f0, f1, f2, zeros, rows)


def _tc_body(q_ref, a_ref, vn_ref, out_ref, *, nt):
    i = pl.program_id(1)
    a3 = a_ref[0, 0:3]
    vnp = vn_ref[0]
    vn3 = vnp[0:3] + vnp[8:11]
    q = q_ref[0]
    dx = q[:, 0:1] - a3[0:1, :]
    dy = q[:, 1:2] - a3[1:2, :]
    dz = q[:, 2:3] - a3[2:3, :]
    d2 = dx * dx + dy * dy + dz * dz
    m = jnp.min(d2, axis=1, keepdims=True)
    sel = jnp.where(d2 == m, 1.0, 0.0)
    an_t = jnp.concatenate([a3, vn3], axis=0)
    ansel = lax.dot_general(sel, an_t, (((1,), (1,)), ((), ())),
                            preferred_element_type=jnp.float32)
    diff = q[:, 0:3] - ansel[:, 0:3]
    dotv = jnp.sum(diff * ansel[:, 3:6], axis=1, keepdims=True)
    l2 = jnp.sqrt(m + 1e-20)
    coll = jnp.where((l2 <= 0.5) & (dotv < 0.0), 1.0, 0.0)
    cnt = jnp.sum(coll)

    @pl.when(i == 0)
    def _():
        out_ref[0, 0, 0] = 0.0
        out_ref[0, 1, 0] = 0.0

    @pl.when(i < nt // 2)
    def _():
        out_ref[0, 0, 0] += cnt

    @pl.when(i >= nt // 2)
    def _():
        out_ref[0, 1, 0] += cnt


def _tc_collisions(q, a_t, vnp):
    bs, ng = q.shape[0], q.shape[1]
    nt = ng // _TQ
    return pl.pallas_call(
        functools.partial(_tc_body, nt=nt),
        grid=(bs, nt),
        in_specs=[
            pl.BlockSpec((1, _TQ, 8), lambda b, i: (b, i, 0)),
            pl.BlockSpec((1, 8, _VPT), lambda b, i: (b, 0, 0)),
            pl.BlockSpec((1, 16, _VPT), lambda b, i: (b, 0, 0)),
        ],
        out_specs=pl.BlockSpec((1, 2, 1), lambda b, i: (b, 0, 0),
                               memory_space=pltpu.SMEM),
        out_shape=jax.ShapeDtypeStruct((bs, 2, 1), jnp.float32),
    )(q, a_t, vnp)[:, :, 0]


def kernel(pred, target, indices, indices_type, faces, h_state, h_faces):
    bs, ng = pred.shape[0], pred.shape[1]
    nh = h_state.shape[1]

    q = jnp.concatenate(
        [pred.astype(jnp.float32), jnp.zeros((bs, ng, 5), jnp.float32)],
        axis=-1)
    a = jnp.transpose(h_state[:, :, :3].astype(jnp.float32), (0, 2, 1))
    a_pad = jnp.pad(a, ((0, 0), (0, 0), (0, _VPT - nh)),
                    constant_values=1e4)
    a_t = jnp.pad(a_pad, ((0, 0), (0, 5), (0, 0)))
    v_pad = jnp.pad(a, ((0, 0), (0, 0), (0, _VP - nh)))
    vx = v_pad[:, 0].reshape(-1)
    vy = v_pad[:, 1].reshape(-1)
    vz = v_pad[:, 2].reshape(-1)
    f_t = jnp.transpose(h_faces, (0, 2, 1)).astype(jnp.int32)
    f_t = jnp.pad(f_t, ((0, 0), (0, 0), (0, _FP - f_t.shape[2])),
                  constant_values=nh)
    f0 = f_t[:, 0].reshape(-1)
    f1 = f_t[:, 1].reshape(-1)
    f2 = f_t[:, 2].reshape(-1)

    vparts = _sc_normals(vx, vy, vz, f0, f1, f2)
    vparts = vparts.reshape(bs, 2, 3, _VP)[:, :, :, :_VPT]
    vparts = jnp.pad(vparts, ((0, 0), (0, 0), (0, 5), (0, 0)))
    vnp = vparts.reshape(bs, 16, _VPT)

    counts = _tc_collisions(q, a_t, vnp)

    total = indices[:, -1].astype(jnp.float32)
    first, second = counts[:, 0], counts[:, 1]
    g2h = jnp.mean((first + second) / total)
    top = jnp.mean(first * indices_type[:, 0, 0] / total)
    bottom = jnp.mean(second * indices_type[:, 1, 1] / total)
    return jnp.stack([bottom, g2h, top])
